# baseline (speedup 1.0000x reference)
import jax
import jax.numpy as jnp
from jax.experimental import pallas as pl
from jax.experimental.pallas import tpu as pltpu

_D = 1024
_SCALE = float(_D) ** 0.5
_BS = 512


def _ppl_kernel(seq_ref, pe_ref, out_ref):
    out_ref[...] = seq_ref[...] * _SCALE + pe_ref[...]


@jax.jit
def _run(sequence, pe):
    batch, seq_len, d = sequence.shape
    rows = batch * seq_len
    seq2d = sequence.reshape(rows, d)
    pe2d = pe.reshape(pe.shape[1], d)
    period = seq_len // _BS
    out = pl.pallas_call(
        _ppl_kernel,
        grid=(rows // _BS,),
        in_specs=[
            pl.BlockSpec((_BS, d), lambda i: (i, 0)),
            pl.BlockSpec((_BS, d), lambda i: (i % period, 0)),
        ],
        out_specs=pl.BlockSpec((_BS, d), lambda i: (i, 0)),
        out_shape=jax.ShapeDtypeStruct((rows, d), sequence.dtype),
        compiler_params=pltpu.CompilerParams(
            dimension_semantics=("parallel",),
        ),
    )(seq2d, pe2d)
    return out.reshape(batch, seq_len, d)


def kernel(sequence, pe, training, mask):
    del training, mask
    return _run(sequence, pe)


# speedup vs baseline: 1.0296x; 1.0296x over previous
import jax
import jax.numpy as jnp
from jax.experimental import pallas as pl
from jax.experimental.pallas import tpu as pltpu

_D = 1024
_SCALE = float(_D) ** 0.5
_BS = 1024


def _ppl_kernel(seq_ref, pe_ref, out_ref):
    out_ref[...] = seq_ref[...] * _SCALE + pe_ref[...]


@jax.jit
def _run(sequence, pe):
    batch, seq_len, d = sequence.shape
    rows = batch * seq_len
    seq2d = sequence.reshape(rows, d)
    pe2d = pe.reshape(pe.shape[1], d)
    period = seq_len // _BS
    out = pl.pallas_call(
        _ppl_kernel,
        grid=(rows // _BS,),
        in_specs=[
            pl.BlockSpec((_BS, d), lambda i: (i, 0)),
            pl.BlockSpec((_BS, d), lambda i: (i % period, 0)),
        ],
        out_specs=pl.BlockSpec((_BS, d), lambda i: (i, 0)),
        out_shape=jax.ShapeDtypeStruct((rows, d), sequence.dtype),
        compiler_params=pltpu.CompilerParams(
            dimension_semantics=("parallel",),
        ),
    )(seq2d, pe2d)
    return out.reshape(batch, seq_len, d)


def kernel(sequence, pe, training, mask):
    del training, mask
    return _run(sequence, pe)


# speedup vs baseline: 1.4206x; 1.3797x over previous
import jax
import jax.numpy as jnp
from jax.experimental import pallas as pl
from jax.experimental.pallas import tpu as pltpu

_D = 1024
_SCALE = float(_D) ** 0.5
_BS = 2048


def _ppl_kernel(seq_ref, pe_ref, out_ref):
    out_ref[...] = seq_ref[...] * _SCALE + pe_ref[...]


@jax.jit
def _run(sequence, pe):
    batch, seq_len, d = sequence.shape
    rows = batch * seq_len
    seq2d = sequence.reshape(rows, d)
    pe2d = pe.reshape(pe.shape[1], d)
    period = seq_len // _BS
    out = pl.pallas_call(
        _ppl_kernel,
        grid=(rows // _BS,),
        in_specs=[
            pl.BlockSpec((_BS, d), lambda i: (i, 0)),
            pl.BlockSpec((_BS, d), lambda i: (i % period, 0)),
        ],
        out_specs=pl.BlockSpec((_BS, d), lambda i: (i, 0)),
        out_shape=jax.ShapeDtypeStruct((rows, d), sequence.dtype),
        compiler_params=pltpu.CompilerParams(
            dimension_semantics=("parallel",),
        ),
    )(seq2d, pe2d)
    return out.reshape(batch, seq_len, d)


def kernel(sequence, pe, training, mask):
    del training, mask
    return _run(sequence, pe)
